# Initial kernel scaffold; baseline (speedup 1.0000x reference)
import jax
import jax.numpy as jnp
from jax.experimental import pallas as pl


def kernel(h0, h1, edge_index, r, basis_0_0, basis_0_1, basis_1_0, basis_1_1, p00_W1, p00_b1, p00_g1, p00_be1, p00_W2, p00_b2, p00_g2, p00_be2, p00_W3, p00_b3, p01_W1, p01_b1, p01_g1, p01_be1, p01_W2, p01_b2, p01_g2, p01_be2, p01_W3, p01_b3, p10_W1, p10_b1, p10_g1, p10_be1, p10_W2, p10_b2, p10_g2, p10_be2, p10_W3, p10_b3, p11_W1, p11_b1, p11_g1, p11_be1, p11_W2, p11_b2, p11_g2, p11_be2, p11_W3, p11_b3):
    raise NotImplementedError("write your pallas kernel here")


# speedup vs baseline: 3.4435x; 3.4435x over previous
import numpy as np
import jax
import jax.numpy as jnp
from jax.experimental import pallas as pl

_N = 10000
_E = 160000
_M = 8
_EB = 2000
_GRID = _E // _EB
_EPS = 1e-5
_HI = jax.lax.Precision.HIGHEST


def _iota2(shape, dim):
    return jax.lax.broadcasted_iota(jnp.int32, shape, dim)


def _group_mat(nl, na, g):
    return (_iota2((nl, na), 0) // g == _iota2((nl, na), 1)).astype(jnp.float32)


def _rep_mat():
    return (_iota2((8, 24), 1) // 3 == _iota2((8, 24), 0)).astype(jnp.float32)


def _sel_mat(i):
    return (_iota2((24, 8), 0) == 3 * _iota2((24, 8), 1) + i).astype(jnp.float32)


def _place_mat(j):
    return (_iota2((8, 24), 1) == 3 * _iota2((8, 24), 0) + j).astype(jnp.float32)


def _dot(a, b):
    return jnp.dot(a, b, precision=_HI, preferred_element_type=jnp.float32)


def _rstats_kernel(r_ref, o_ref):
    i = pl.program_id(0)

    @pl.when(i == 0)
    def _init():
        o_ref[...] = jnp.zeros_like(o_ref)

    r = r_ref[...]
    s = jnp.sum(r)
    s2 = jnp.sum(r * r)
    row = jnp.concatenate(
        [jnp.full((1, 128), s, jnp.float32), jnp.full((1, 128), s2, jnp.float32)],
        axis=0)
    o_ref[...] = o_ref[...] + row


def _zstats_kernel(r_ref, a1_ref, c1_ref, a200, a201, a210, a211, o_ref):
    i = pl.program_id(0)

    @pl.when(i == 0)
    def _init():
        o_ref[...] = jnp.zeros_like(o_ref)

    r = r_ref[...]
    rows = []
    for p, a2 in enumerate((a200, a201, a210, a211)):
        y1 = jnp.maximum(_dot(r, a1_ref[p:p + 1, :]) + c1_ref[p:p + 1, :], 0.0)
        z = _dot(y1, a2[...])
        rows.append(jnp.sum(z, axis=0, keepdims=True))
        rows.append(jnp.sum(z * z, axis=0, keepdims=True))
    o_ref[...] = o_ref[...] + jnp.concatenate(rows, axis=0)


def _msg_kernel(r_ref, s0_ref, s1_ref, b00_ref, b01_ref, b10_ref, b11_ref,
                a1_ref, c1_ref, s2_ref, t2_ref,
                a200, a201, a210, a211,
                a300, a301, a310, a311,
                b300, b301, b310, b311, o_ref):
    r = r_ref[...]
    s0 = s0_ref[...]
    s1 = s1_ref[...]
    b00 = b00_ref[...]
    b01 = b01_ref[...]
    b10 = b10_ref[...]
    b11 = b11_ref[...]
    G8 = _group_mat(64, 8, 8)
    G24 = _group_mat(192, 8, 24)
    P = _rep_mat()

    def radial(p, a2, a3, b3):
        y1 = jnp.maximum(_dot(r, a1_ref[p:p + 1, :]) + c1_ref[p:p + 1, :], 0.0)
        z = _dot(y1, a2[...])
        y2 = jnp.maximum(z * s2_ref[p:p + 1, :] + t2_ref[p:p + 1, :], 0.0)
        return _dot(y2, a3[...]) + b3[...]

    R00 = radial(0, a200, a300, b300)
    R01 = radial(1, a201, a301, b301)
    R10 = radial(2, a210, a310, b310)
    R11 = radial(3, a211, a311, b311)

    s0t = jnp.tile(s0, (1, 8))

    s1_i = [_dot(s1, _sel_mat(i)) for i in range(3)]
    u = (s1_i[0] * b10[:, 0:1] + s1_i[1] * b10[:, 1:2]
         + s1_i[2] * b10[:, 2:3])

    msg0 = _dot(R00 * s0t, G8) * b00 + _dot(R10 * jnp.tile(u, (1, 8)), G8)

    q01 = _dot(R01 * s0t, G8)
    out1 = _dot(q01, P) * jnp.tile(b01, (1, 8))

    s1rep = [_dot(si, P) for si in s1_i]
    for j in range(3):
        tf = (s1rep[0] * jnp.tile(b11[:, 9 * j + 0:9 * j + 3], (1, 8))
              + s1rep[1] * jnp.tile(b11[:, 9 * j + 3:9 * j + 6], (1, 8))
              + s1rep[2] * jnp.tile(b11[:, 9 * j + 6:9 * j + 9], (1, 8)))
        mj = _dot(R11 * jnp.tile(tf, (1, 8)), G24)
        out1 = out1 + _dot(mj, _place_mat(j))

    o_ref[...] = jnp.concatenate([msg0, out1], axis=1)


def kernel(h0, h1, edge_index, r, basis_0_0, basis_0_1, basis_1_0, basis_1_1,
           p00_W1, p00_b1, p00_g1, p00_be1, p00_W2, p00_b2, p00_g2, p00_be2, p00_W3, p00_b3,
           p01_W1, p01_b1, p01_g1, p01_be1, p01_W2, p01_b2, p01_g2, p01_be2, p01_W3, p01_b3,
           p10_W1, p10_b1, p10_g1, p10_be1, p10_W2, p10_b2, p10_g2, p10_be2, p10_W3, p10_b3,
           p11_W1, p11_b1, p11_g1, p11_be1, p11_W2, p11_b2, p11_g2, p11_be2, p11_W3, p11_b3):
    src, dst = edge_index[0], edge_index[1]
    b00 = basis_0_0.reshape(_E, 1)
    b01 = basis_0_1.reshape(_E, 3)
    b10 = basis_1_0.reshape(_E, 3)
    b11 = basis_1_1.reshape(_E, 27)
    s0 = jnp.take(h0.reshape(_N, _M), src, axis=0)
    s1 = jnp.take(h1.reshape(_N, 3 * _M), src, axis=0)

    edge_spec = lambda d: pl.BlockSpec((_EB, d), lambda i: (i, 0))
    const_spec = lambda s: pl.BlockSpec(s, lambda i: (0, 0))

    rst = pl.pallas_call(
        _rstats_kernel,
        grid=(_GRID,),
        in_specs=[edge_spec(1)],
        out_specs=const_spec((2, 128)),
        out_shape=jax.ShapeDtypeStruct((2, 128), jnp.float32),
    )(r)
    mean_r = rst[0, 0] / _E
    var_r = rst[1, 0] / _E - mean_r * mean_r

    W1s = (p00_W1, p01_W1, p10_W1, p11_W1)
    g1s = (p00_g1, p01_g1, p10_g1, p11_g1)
    be1s = (p00_be1, p01_be1, p10_be1, p11_be1)
    a1_rows, c1_rows = [], []
    for w1, g1, be1 in zip(W1s, g1s, be1s):
        w = w1[:, 0]
        a = g1 * w * jax.lax.rsqrt(w * w * var_r + _EPS)
        a1_rows.append(a)
        c1_rows.append(be1 - a * mean_r)
    A1 = jnp.stack(a1_rows)
    C1 = jnp.stack(c1_rows)
    A2s = [w.T for w in (p00_W2, p01_W2, p10_W2, p11_W2)]

    zst = pl.pallas_call(
        _zstats_kernel,
        grid=(_GRID,),
        in_specs=[edge_spec(1), const_spec((4, 32)), const_spec((4, 32))]
        + [const_spec((32, 32))] * 4,
        out_specs=const_spec((8, 32)),
        out_shape=jax.ShapeDtypeStruct((8, 32), jnp.float32),
    )(r, A1, C1, *A2s)
    g2s = (p00_g2, p01_g2, p10_g2, p11_g2)
    be2s = (p00_be2, p01_be2, p10_be2, p11_be2)
    s2_rows, t2_rows = [], []
    for p, (g2, be2) in enumerate(zip(g2s, be2s)):
        mu = zst[2 * p] / _E
        var = zst[2 * p + 1] / _E - mu * mu
        s2 = g2 * jax.lax.rsqrt(var + _EPS)
        s2_rows.append(s2)
        t2_rows.append(be2 - s2 * mu)
    S2 = jnp.stack(s2_rows)
    T2 = jnp.stack(t2_rows)

    A3s = [w.T for w in (p00_W3, p01_W3, p10_W3, p11_W3)]
    B3s = [b.reshape(1, -1) for b in (p00_b3, p01_b3, p10_b3, p11_b3)]

    msg = pl.pallas_call(
        _msg_kernel,
        grid=(_GRID,),
        in_specs=[edge_spec(1), edge_spec(8), edge_spec(24), edge_spec(1),
                  edge_spec(3), edge_spec(3), edge_spec(27)]
        + [const_spec((4, 32))] * 4
        + [const_spec((32, 32))] * 4
        + [const_spec((32, 64))] * 3 + [const_spec((32, 192))]
        + [const_spec((1, 64))] * 3 + [const_spec((1, 192))],
        out_specs=edge_spec(32),
        out_shape=jax.ShapeDtypeStruct((_E, 32), jnp.float32),
    )(r, s0, s1, b00, b01, b10, b11, A1, C1, S2, T2, *A2s, *A3s, *B3s)

    deg = jax.ops.segment_sum(jnp.ones((_E,), jnp.float32), dst,
                              num_segments=_N)
    deg = jnp.maximum(deg, 1.0)
    out = jax.ops.segment_sum(msg, dst, num_segments=_N) / deg[:, None]
    return out
